# baseline (speedup 1.0000x reference)
import jax
import jax.numpy as jnp
from jax import lax
from jax.experimental import pallas as pl
from jax.experimental.pallas import tpu as pltpu
from jax.experimental.pallas import tpu_sc as plsc

B = 1024
NUM_WRITES = 2
NUM_READS = 4
MEMORY_SIZE = 8192

ROWS_TC = 64
BT = 576
NC = 2
NS = 16
NW = NC * NS
B_SC = B - BT
RPW = B_SC // NW
CM = 4096
NCHUNK = MEMORY_SIZE // CM
STEPS = RPW * NCHUNK
LANES = 16


def _tc_body(ww_ref, fg_ref, rw_ref, prev_ref, out_ref):
    prev = prev_ref[...]
    q = (1.0 - ww_ref[:, 0, :]) * (1.0 - ww_ref[:, 1, :])
    usage = 1.0 - (1.0 - prev) * q
    fg = fg_ref[...]
    phi = (1.0 - fg[:, 0, None] * rw_ref[:, 0, :])
    phi = phi * (1.0 - fg[:, 1, None] * rw_ref[:, 1, :])
    phi = phi * (1.0 - fg[:, 2, None] * rw_ref[:, 2, :])
    phi = phi * (1.0 - fg[:, 3, None] * rw_ref[:, 3, :])
    out_ref[...] = usage * phi


def _tc_part(write_weights, free_gate, read_weights, prev_usage):
    return pl.pallas_call(
        _tc_body,
        grid=(BT // ROWS_TC,),
        in_specs=[
            pl.BlockSpec((ROWS_TC, NUM_WRITES, MEMORY_SIZE), lambda i: (i, 0, 0)),
            pl.BlockSpec((ROWS_TC, NUM_READS), lambda i: (i, 0)),
            pl.BlockSpec((ROWS_TC, NUM_READS, MEMORY_SIZE), lambda i: (i, 0, 0)),
            pl.BlockSpec((ROWS_TC, MEMORY_SIZE), lambda i: (i, 0)),
        ],
        out_specs=pl.BlockSpec((ROWS_TC, MEMORY_SIZE), lambda i: (i, 0)),
        out_shape=jax.ShapeDtypeStruct((B, MEMORY_SIZE), jnp.float32),
        compiler_params=pltpu.CompilerParams(
            dimension_semantics=("arbitrary",),
        ),
    )(write_weights, free_gate, read_weights, prev_usage)


def _sc_body(ww_hbm, fg_hbm, rw_hbm, prev_hbm, out_hbm,
             w0b, w1b, r0b, r1b, r2b, r3b, pvb, ob, fgb,
             sem_in0, sem_in1, sem_out0, sem_out1, sem_fg):
    wid = lax.axis_index("s") * NC + lax.axis_index("c")
    obase = wid * RPW
    base = BT + obase

    in_sems = (sem_in0, sem_in1)
    out_sems = (sem_out0, sem_out1)
    inbufs = (w0b, w1b, r0b, r1b, r2b, r3b, pvb)

    pltpu.async_copy(fg_hbm.at[pl.ds(base * 128, RPW * 128)], fgb, sem_fg).wait()

    def issue_in(row, coff, p):
        sem = in_sems[p]
        srcs = (
            ww_hbm.at[row, 0, pl.ds(coff, CM)],
            ww_hbm.at[row, 1, pl.ds(coff, CM)],
            rw_hbm.at[row, 0, pl.ds(coff, CM)],
            rw_hbm.at[row, 1, pl.ds(coff, CM)],
            rw_hbm.at[row, 2, pl.ds(coff, CM)],
            rw_hbm.at[row, 3, pl.ds(coff, CM)],
            prev_hbm.at[row, pl.ds(coff, CM)],
        )
        for src, buf in zip(srcs, inbufs):
            pltpu.async_copy(src, buf.at[p], sem)

    def wait_in(p):
        for buf in inbufs:
            pltpu.make_async_copy(prev_hbm.at[0, pl.ds(0, CM)], buf.at[p],
                                  in_sems[p]).wait()

    def wait_out(p):
        pltpu.make_async_copy(ob.at[p], out_hbm.at[0, pl.ds(0, CM)],
                              out_sems[p]).wait()

    def do_compute(p, row_local):
        fg_row = fgb[pl.ds(row_local * 128, LANES)]
        fvec = [jnp.full((LANES,), fg_row[k], jnp.float32)
                for k in range(NUM_READS)]

        @plsc.parallel_loop(0, CM // LANES, unroll=8)
        def chunk_body(i):
            o = i * LANES
            pv = pvb[p, pl.ds(o, LANES)]
            q = (1.0 - w0b[p, pl.ds(o, LANES)]) * (1.0 - w1b[p, pl.ds(o, LANES)])
            u = 1.0 - (1.0 - pv) * q
            phi = 1.0 - fvec[0] * r0b[p, pl.ds(o, LANES)]
            phi = phi * (1.0 - fvec[1] * r1b[p, pl.ds(o, LANES)])
            phi = phi * (1.0 - fvec[2] * r2b[p, pl.ds(o, LANES)])
            phi = phi * (1.0 - fvec[3] * r3b[p, pl.ds(o, LANES)])
            ob[p, pl.ds(o, LANES)] = u * phi

    issue_in(base, 0, 0)

    def step(t, carry):
        tn = t + 1
        row_local = t // NCHUNK
        row = base + row_local
        coff = (t % NCHUNK) * CM
        rown = base + tn // NCHUNK
        coffn = (tn % NCHUNK) * CM
        p0 = (t % 2) == 0

        @pl.when(jnp.logical_and(tn < STEPS, (tn % 2) == 0))
        def _():
            issue_in(rown, coffn, 0)

        @pl.when(jnp.logical_and(tn < STEPS, (tn % 2) == 1))
        def _():
            issue_in(rown, coffn, 1)

        def run_parity(p):
            wait_in(p)

            @pl.when(t >= 2)
            def _():
                wait_out(p)

            do_compute(p, row_local)
            pltpu.async_copy(ob.at[p], out_hbm.at[obase + row_local, pl.ds(coff, CM)],
                             out_sems[p])

        @pl.when(p0)
        def _():
            run_parity(0)

        @pl.when(jnp.logical_not(p0))
        def _():
            run_parity(1)

        return carry

    lax.fori_loop(0, STEPS, step, 0)
    wait_out(0)
    wait_out(1)


def _sc_part(write_weights, fg_pad, read_weights, prev_usage):
    mesh = plsc.VectorSubcoreMesh(core_axis_name="c", subcore_axis_name="s")
    f = pl.kernel(
        _sc_body,
        out_type=jax.ShapeDtypeStruct((B_SC, MEMORY_SIZE), jnp.float32),
        mesh=mesh,
        scratch_types=[
            pltpu.VMEM((2, CM), jnp.float32),
            pltpu.VMEM((2, CM), jnp.float32),
            pltpu.VMEM((2, CM), jnp.float32),
            pltpu.VMEM((2, CM), jnp.float32),
            pltpu.VMEM((2, CM), jnp.float32),
            pltpu.VMEM((2, CM), jnp.float32),
            pltpu.VMEM((2, CM), jnp.float32),
            pltpu.VMEM((2, CM), jnp.float32),
            pltpu.VMEM((RPW * 128,), jnp.float32),
            pltpu.SemaphoreType.DMA,
            pltpu.SemaphoreType.DMA,
            pltpu.SemaphoreType.DMA,
            pltpu.SemaphoreType.DMA,
            pltpu.SemaphoreType.DMA,
        ],
    )
    return f(write_weights, fg_pad, read_weights, prev_usage)


def _stitch_body(full_ref, sc_ref, out_ref):
    out_ref[...] = sc_ref[...]


def _stitch(tc_out, sc_out):
    return pl.pallas_call(
        _stitch_body,
        grid=(B_SC // ROWS_TC,),
        in_specs=[
            pl.BlockSpec(memory_space=pl.ANY),
            pl.BlockSpec((ROWS_TC, MEMORY_SIZE), lambda i: (i, 0)),
        ],
        out_specs=pl.BlockSpec((ROWS_TC, MEMORY_SIZE),
                               lambda i: (BT // ROWS_TC + i, 0)),
        out_shape=jax.ShapeDtypeStruct((B, MEMORY_SIZE), jnp.float32),
        input_output_aliases={0: 0},
        compiler_params=pltpu.CompilerParams(
            dimension_semantics=("arbitrary",),
        ),
    )(tc_out, sc_out)


@jax.jit
def kernel(write_weights, free_gate, read_weights, prev_usage):
    fg_lin = jnp.pad(free_gate, ((0, 0), (0, 128 - NUM_READS))).reshape(B * 128)
    sc_out = _sc_part(write_weights, fg_lin, read_weights, prev_usage)
    tc_out = _tc_part(write_weights, free_gate, read_weights, prev_usage)
    return _stitch(tc_out, sc_out)


# speedup vs baseline: 1.1821x; 1.1821x over previous
import jax
import jax.numpy as jnp
from jax.experimental import pallas as pl
from jax.experimental.pallas import tpu as pltpu

B = 1024
NUM_WRITES = 2
NUM_READS = 4
MEMORY_SIZE = 8192

ROWS = 64


def _body(ww_ref, fg_ref, rw_ref, prev_ref, out_ref):
    prev = prev_ref[...]
    q = (1.0 - ww_ref[:, 0, :]) * (1.0 - ww_ref[:, 1, :])
    usage = 1.0 - (1.0 - prev) * q
    fg = fg_ref[...]
    phi = (1.0 - fg[:, 0, None] * rw_ref[:, 0, :])
    phi = phi * (1.0 - fg[:, 1, None] * rw_ref[:, 1, :])
    phi = phi * (1.0 - fg[:, 2, None] * rw_ref[:, 2, :])
    phi = phi * (1.0 - fg[:, 3, None] * rw_ref[:, 3, :])
    out_ref[...] = usage * phi


def kernel(write_weights, free_gate, read_weights, prev_usage):
    MC = MEMORY_SIZE // 2
    grid = (B // ROWS, 2)
    return pl.pallas_call(
        _body,
        grid=grid,
        in_specs=[
            pl.BlockSpec((ROWS, NUM_WRITES, MC), lambda i, j: (i, 0, j)),
            pl.BlockSpec((ROWS, NUM_READS), lambda i, j: (i, 0)),
            pl.BlockSpec((ROWS, NUM_READS, MC), lambda i, j: (i, 0, j)),
            pl.BlockSpec((ROWS, MC), lambda i, j: (i, j)),
        ],
        out_specs=pl.BlockSpec((ROWS, MC), lambda i, j: (i, j)),
        out_shape=jax.ShapeDtypeStruct((B, MEMORY_SIZE), jnp.float32),
        compiler_params=pltpu.CompilerParams(
            dimension_semantics=("arbitrary", "arbitrary"),
        ),
    )(write_weights, free_gate, read_weights, prev_usage)
